# Initial kernel scaffold; baseline (speedup 1.0000x reference)
import jax
import jax.numpy as jnp
from jax.experimental import pallas as pl


def kernel(embeddings, prototypes):
    raise NotImplementedError("write your pallas kernel here")


# speedup vs baseline: 1.0746x; 1.0746x over previous
import jax
import jax.numpy as jnp
from jax.experimental import pallas as pl

_B = 16384
_K = 8192
_D = 32
_TB = 256


def _assign_kernel(emb_ref, proto_ref, out_ref):
    emb = emb_ref[...]
    en = emb / jnp.clip(
        jnp.sqrt(jnp.sum(emb * emb, axis=1, keepdims=True)), 1e-12)
    proto = proto_ref[...]
    pn = proto / jnp.clip(
        jnp.sqrt(jnp.sum(proto * proto, axis=1, keepdims=True)), 1e-12)
    sims = jax.lax.dot_general(
        en, pn, (((1,), (1,)), ((), ())),
        preferred_element_type=jnp.float32)
    out_ref[0, :] = jnp.argmax(sims, axis=1).astype(jnp.int32)


def kernel(embeddings, prototypes):
    out = pl.pallas_call(
        _assign_kernel,
        grid=(_B // _TB,),
        in_specs=[
            pl.BlockSpec((_TB, _D), lambda i: (i, 0)),
            pl.BlockSpec((_K, _D), lambda i: (0, 0)),
        ],
        out_specs=pl.BlockSpec((1, _TB), lambda i: (0, i)),
        out_shape=jax.ShapeDtypeStruct((1, _B), jnp.int32),
    )(embeddings, prototypes)
    return out[0]


# speedup vs baseline: 1.8166x; 1.6906x over previous
import jax
import jax.numpy as jnp
from jax.experimental import pallas as pl
from jax.experimental.pallas import tpu as pltpu

_B = 16384
_K = 8192
_D = 32
_TB = 256
_LANES = 128
_KC = 1024


def _assign_kernel(emb_ref, proto_ref, out_ref, pn_ref):
    i = pl.program_id(0)

    @pl.when(i == 0)
    def _():
        proto = proto_ref[...]
        pn_ref[...] = proto / jnp.clip(
            jnp.sqrt(jnp.sum(proto * proto, axis=1, keepdims=True)), 1e-12)

    emb = emb_ref[...]
    en = emb / jnp.clip(
        jnp.sqrt(jnp.sum(emb * emb, axis=1, keepdims=True)), 1e-12)

    acc_v = jnp.full((_TB, _LANES), -jnp.inf, jnp.float32)
    acc_i = jnp.zeros((_TB, _LANES), jnp.int32)
    for c in range(_K // _KC):
        sims = jax.lax.dot_general(
            en, pn_ref[c * _KC:(c + 1) * _KC, :],
            (((1,), (1,)), ((), ())),
            preferred_element_type=jnp.float32)
        for jj in range(_KC // _LANES):
            j = c * (_KC // _LANES) + jj
            v = sims[:, jj * _LANES:(jj + 1) * _LANES]
            m = v > acc_v
            acc_v = jnp.where(m, v, acc_v)
            acc_i = jnp.where(m, j, acc_i)
    rowmax = jnp.max(acc_v, axis=1, keepdims=True)
    lane = jax.lax.broadcasted_iota(jnp.int32, (_TB, _LANES), 1)
    g = acc_i * _LANES + lane
    cand = jnp.where(acc_v == rowmax, g, jnp.int32(2 ** 30))
    out_ref[0, :] = jnp.min(cand, axis=1)


def kernel(embeddings, prototypes):
    out = pl.pallas_call(
        _assign_kernel,
        grid=(_B // _TB,),
        in_specs=[
            pl.BlockSpec((_TB, _D), lambda i: (i, 0)),
            pl.BlockSpec((_K, _D), lambda i: (0, 0)),
        ],
        out_specs=pl.BlockSpec((1, _TB), lambda i: (0, i)),
        out_shape=jax.ShapeDtypeStruct((1, _B), jnp.int32),
        scratch_shapes=[pltpu.VMEM((_K, _D), jnp.float32)],
    )(embeddings, prototypes)
    return out[0]
